# baseline (speedup 1.0000x reference)
import functools

import jax
import jax.numpy as jnp
from jax import lax
from jax.experimental import pallas as pl
from jax.experimental.pallas import tpu as pltpu
from jax.experimental.pallas import tpu_sc as plsc

N = 10000
E = 320000
D = 128
D2 = 64

NC = 2
NS = 16
CHUNK = 125
ECH = E // CHUNK
CPW_F = ECH // NS
CPW_E = ECH // (NC * NS)
NP = 10240
RPW = NP // NS
ZROWS = 64

_P = lax.Precision.DEFAULT


def _dot(a, b):
    return lax.dot_general(a, b, (((1,), (0,)), ((), ())),
                           precision=_P, preferred_element_type=jnp.float32)


def _elu(x):
    return jnp.where(x > 0, x, jnp.exp(x) - 1.0)


def _tc_mm1(x, w0, w1, w2, w3):
    def body(x_ref, w0_ref, w1_ref, w2_ref, w3_ref, o0, o1, o2, o3):
        xb = x_ref[...]
        o0[...] = _dot(xb, w0_ref[...])
        o1[...] = _dot(xb, w1_ref[...])
        o2[...] = _dot(xb, w2_ref[...])
        o3[...] = _dot(xb, w3_ref[...])

    return pl.pallas_call(
        body,
        grid=(10,),
        in_specs=[pl.BlockSpec((N // 10, D), lambda i: (i, 0))] +
                 [pl.BlockSpec((D, D2), lambda i: (0, 0))] * 4,
        out_specs=[pl.BlockSpec((N // 10, D2), lambda i: (i, 0))] * 4,
        out_shape=[jax.ShapeDtypeStruct((N, D2), jnp.float32)] * 4,
    )(x, w0, w1, w2, w3)


def _tc_mid(accs, cnt_a, cnt_b, w2):
    def body(a0, a1, a2, a3, ca_ref, cb_ref, w_ref, o_ref):
        cnt = ca_ref[:, 0:1] + cb_ref[:, 0:1]
        denom = jnp.maximum(cnt, 1.0)
        acc = _dot(_elu(a0[...] / denom), w_ref[pl.ds(0, D2), :])
        acc += _dot(_elu(a1[...] / denom), w_ref[pl.ds(D2, D2), :])
        acc += _dot(_elu(a2[...] / denom), w_ref[pl.ds(2 * D2, D2), :])
        acc += _dot(_elu(a3[...] / denom), w_ref[pl.ds(3 * D2, D2), :])
        o_ref[...] = acc

    return pl.pallas_call(
        body,
        grid=(10,),
        in_specs=[pl.BlockSpec((NP // 10, D2), lambda i: (i, 0))] * 4 +
                 [pl.BlockSpec((NP // 10, 16), lambda i: (i, 0))] * 2 +
                 [pl.BlockSpec((4 * D2, D2), lambda i: (0, 0))],
        out_specs=[pl.BlockSpec((NP // 10, D2), lambda i: (i, 0))],
        out_shape=[jax.ShapeDtypeStruct((NP, D2), jnp.float32)],
    )(*accs, cnt_a, cnt_b, w2)[0]


def _tc_final(p0, p1, cnt_a, cnt_b):
    def body(p0_ref, p1_ref, ca_ref, cb_ref, o_ref):
        cnt = ca_ref[:, 0:1] + cb_ref[:, 0:1]
        denom = jnp.maximum(cnt, 1.0)
        o_ref[...] = (p0_ref[...] + p1_ref[...]) / denom

    return pl.pallas_call(
        body,
        grid=(10,),
        in_specs=[pl.BlockSpec((NP // 10, D2), lambda i: (i, 0))] * 2 +
                 [pl.BlockSpec((NP // 10, 16), lambda i: (i, 0))] * 2,
        out_specs=[pl.BlockSpec((NP // 10, D2), lambda i: (i, 0))],
        out_shape=[jax.ShapeDtypeStruct((NP, D2), jnp.float32)],
    )(p0, p1, cnt_a, cnt_b)[0]


NBUF = 5
LOOK = 2


def _edge_pipeline(t_hbm, acc_sp, dst_v, src_v, bufs, gsems, ssems, nchunks):
    lag = NBUF - LOOK
    assert nchunks % NBUF == 0
    for j in range(LOOK):
        pltpu.async_copy(t_hbm.at[dst_v.at[j]], bufs[j], gsems[j])

    @pl.loop(0, nchunks // NBUF)
    def _(jj):
        i_base = NBUF * jj
        for b in range(NBUF):
            i = i_base + b
            pltpu.make_async_copy(t_hbm.at[dst_v.at[i]], bufs[b],
                                  gsems[b]).wait()

            @pl.when(i >= lag)
            def _():
                pltpu.make_async_copy(bufs[(b - lag) % NBUF],
                                      acc_sp.at[src_v.at[i - lag]],
                                      ssems[(b - lag) % NBUF]).wait()

            @pl.when(i + LOOK < nchunks)
            def _():
                pltpu.async_copy(t_hbm.at[dst_v.at[i + LOOK]],
                                 bufs[(b + LOOK) % NBUF],
                                 gsems[(b + LOOK) % NBUF])

            pltpu.async_copy(bufs[b], acc_sp.at[src_v.at[i]], ssems[b],
                             add=True)

    for j in range(lag):
        i = nchunks - lag + j
        pltpu.make_async_copy(bufs[i % NBUF], acc_sp.at[src_v.at[i]],
                              ssems[i % NBUF]).wait()


def _zero_vmem(buf, rows, cols):
    @pl.loop(0, rows)
    def _(i):
        @pl.loop(0, cols, step=16)
        def _(j):
            buf[i, pl.ds(j, 16)] = jnp.zeros((16,), jnp.float32)


def _zero_stripe(buf, acc_sp, base):
    @pl.loop(0, RPW // ZROWS)
    def _(k):
        pltpu.sync_copy(buf, acc_sp.at[pl.ds(base + k * ZROWS, ZROWS)])


def _sc_counts(src4d):
    mesh = plsc.VectorSubcoreMesh(core_axis_name="c", subcore_axis_name="s")
    out_type = [jax.ShapeDtypeStruct((NP, 16), jnp.float32)] * 2

    @functools.partial(
        pl.kernel, mesh=mesh, out_type=out_type,
        compiler_params=pltpu.CompilerParams(use_tc_tiling_on_sc=False),
        scratch_types=[
            pltpu.VMEM((CPW_E, CHUNK), jnp.int32),
            pltpu.VMEM((ZROWS, 16), jnp.float32),
            pltpu.VMEM((CHUNK, 16), jnp.float32),
            pltpu.VMEM_SHARED((NP, 16), jnp.float32),
            pltpu.SemaphoreType.DMA,
        ],
    )
    def k(srcc_hbm, cnta_hbm, cntb_hbm, srcc_v, z16_v, ones_v, cnt_sp, csem):
        c = lax.axis_index("c")
        s = lax.axis_index("s")
        base = s * RPW

        pltpu.sync_copy(srcc_hbm.at[c, s], srcc_v)
        _zero_vmem(z16_v, ZROWS, 16)
        _zero_stripe(z16_v, cnt_sp, base)

        @pl.loop(0, CHUNK)
        def _(i):
            ones_v[i, pl.ds(0, 16)] = jnp.ones((16,), jnp.float32)

        plsc.subcore_barrier()

        @pl.loop(0, CPW_E // 8)
        def _(f):
            @pl.loop(0, 8)
            def _(j):
                pltpu.async_copy(ones_v, cnt_sp.at[srcc_v.at[f * 8 + j]],
                                 csem, add=True)

            @pl.loop(0, 8)
            def _(j):
                pltpu.make_async_copy(ones_v, cnt_sp.at[srcc_v.at[f * 8 + j]],
                                      csem).wait()

        plsc.subcore_barrier()

        @pl.when(c == 0)
        def _():
            pltpu.sync_copy(cnt_sp.at[pl.ds(base, RPW)],
                            cnta_hbm.at[pl.ds(base, RPW)])

        @pl.when(c == 1)
        def _():
            pltpu.sync_copy(cnt_sp.at[pl.ds(base, RPW)],
                            cntb_hbm.at[pl.ds(base, RPW)])

    return k(src4d)


def _sc_layer1(t1qs, dst3d, src3d):
    mesh = plsc.VectorSubcoreMesh(core_axis_name="c", subcore_axis_name="s")
    out_type = [jax.ShapeDtypeStruct((NP, D2), jnp.float32)] * 4

    @functools.partial(
        pl.kernel, mesh=mesh, out_type=out_type,
        compiler_params=pltpu.CompilerParams(use_tc_tiling_on_sc=False),
        scratch_types=[
            pltpu.VMEM((CPW_F, CHUNK), jnp.int32),
            pltpu.VMEM((CPW_F, CHUNK), jnp.int32),
        ] + [pltpu.VMEM((CHUNK, D2), jnp.float32)] * NBUF +
        [
            pltpu.VMEM((ZROWS, D2), jnp.float32),
            pltpu.VMEM_SHARED((NP, D2), jnp.float32),
        ] + [pltpu.SemaphoreType.DMA] * (2 * NBUF),
    )
    def k(t1q0_hbm, t1q1_hbm, t1q2_hbm, t1q3_hbm, dst_hbm, src_hbm,
          acc0_hbm, acc1_hbm, acc2_hbm, acc3_hbm,
          dst_v, src_v, *scr):
        bufs = list(scr[:NBUF])
        zf_v = scr[NBUF]
        acc_sp = scr[NBUF + 1]
        gsems = list(scr[NBUF + 2:2 * NBUF + 2])
        ssems = list(scr[2 * NBUF + 2:3 * NBUF + 2])
        c = lax.axis_index("c")
        s = lax.axis_index("s")
        base = s * RPW

        _zero_vmem(zf_v, ZROWS, D2)
        _zero_stripe(zf_v, acc_sp, base)

        plsc.subcore_barrier()

        pltpu.sync_copy(dst_hbm.at[s], dst_v)
        pltpu.sync_copy(src_hbm.at[s], src_v)

        def head_pass(t_hbm, out_hbm):
            _edge_pipeline(t_hbm, acc_sp, dst_v, src_v, bufs, gsems,
                           ssems, CPW_F)
            plsc.subcore_barrier()
            pltpu.sync_copy(acc_sp.at[pl.ds(base, RPW)],
                            out_hbm.at[pl.ds(base, RPW)])

        def core_passes(ta_hbm, tb_hbm, outa_hbm, outb_hbm):
            head_pass(ta_hbm, outa_hbm)
            _zero_stripe(zf_v, acc_sp, base)
            plsc.subcore_barrier()
            head_pass(tb_hbm, outb_hbm)

        @pl.when(c == 0)
        def _():
            core_passes(t1q0_hbm, t1q1_hbm, acc0_hbm, acc1_hbm)

        @pl.when(c == 1)
        def _():
            core_passes(t1q2_hbm, t1q3_hbm, acc2_hbm, acc3_hbm)

    return k(*t1qs, dst3d, src3d)


def _sc_layer2(t2, dst4d, src4d):
    mesh = plsc.VectorSubcoreMesh(core_axis_name="c", subcore_axis_name="s")
    out_type = [
        jax.ShapeDtypeStruct((NP, D2), jnp.float32),
        jax.ShapeDtypeStruct((NP, D2), jnp.float32),
    ]

    @functools.partial(
        pl.kernel, mesh=mesh, out_type=out_type,
        compiler_params=pltpu.CompilerParams(use_tc_tiling_on_sc=False),
        scratch_types=[
            pltpu.VMEM((CPW_E, CHUNK), jnp.int32),
            pltpu.VMEM((CPW_E, CHUNK), jnp.int32),
        ] + [pltpu.VMEM((CHUNK, D2), jnp.float32)] * NBUF +
        [
            pltpu.VMEM((ZROWS, D2), jnp.float32),
            pltpu.VMEM_SHARED((NP, D2), jnp.float32),
        ] + [pltpu.SemaphoreType.DMA] * (2 * NBUF),
    )
    def k(t2_hbm, dst_hbm, src_hbm, p0_hbm, p1_hbm,
          dst_v, src_v, *scr):
        bufs = list(scr[:NBUF])
        zf_v = scr[NBUF]
        acc_sp = scr[NBUF + 1]
        gsems = list(scr[NBUF + 2:2 * NBUF + 2])
        ssems = list(scr[2 * NBUF + 2:3 * NBUF + 2])
        c = lax.axis_index("c")
        s = lax.axis_index("s")
        base = s * RPW

        _zero_vmem(zf_v, ZROWS, D2)
        _zero_stripe(zf_v, acc_sp, base)

        plsc.subcore_barrier()

        pltpu.sync_copy(dst_hbm.at[c, s], dst_v)
        pltpu.sync_copy(src_hbm.at[c, s], src_v)
        _edge_pipeline(t2_hbm, acc_sp, dst_v, src_v, bufs, gsems, ssems,
                       CPW_E)

        plsc.subcore_barrier()

        @pl.when(c == 0)
        def _():
            pltpu.sync_copy(acc_sp.at[pl.ds(base, RPW)],
                            p0_hbm.at[pl.ds(base, RPW)])

        @pl.when(c == 1)
        def _():
            pltpu.sync_copy(acc_sp.at[pl.ds(base, RPW)],
                            p1_hbm.at[pl.ds(base, RPW)])

    return k(t2, dst4d, src4d)


def kernel(node_features, edge_index, W1_0, W1_1, W1_2, W1_3, W2_0):
    src = edge_index[0]
    dst = edge_index[1]
    src3d = src.reshape(NS, CPW_F, CHUNK)
    dst3d = dst.reshape(NS, CPW_F, CHUNK)
    src4d = src.reshape(NC, NS, CPW_E, CHUNK)
    dst4d = dst.reshape(NC, NS, CPW_E, CHUNK)

    cnt_a, cnt_b = _sc_counts(src4d)
    t1qs = _tc_mm1(node_features, W1_0, W1_1, W1_2, W1_3)
    a0, a1, a2, a3 = _sc_layer1(t1qs, dst3d, src3d)
    t2 = _tc_mid((a0, a1, a2, a3), cnt_a, cnt_b, W2_0)
    p0, p1 = _sc_layer2(t2, dst4d, src4d)
    return _tc_final(p0, p1, cnt_a, cnt_b)[:N]


# speedup vs baseline: 1.1682x; 1.1682x over previous
import functools

import jax
import jax.numpy as jnp
from jax import lax
from jax.experimental import pallas as pl
from jax.experimental.pallas import tpu as pltpu
from jax.experimental.pallas import tpu_sc as plsc

N = 10000
E = 320000
D = 128
D2 = 64

NC = 2
NS = 16
CHUNK = 125
ECH = E // CHUNK
CPW_F = ECH // NS
CPW_E = ECH // (NC * NS)
NP = 10240
RPW = NP // NS
ZROWS = 64

_P = lax.Precision.DEFAULT


def _dot(a, b):
    return lax.dot_general(a, b, (((1,), (0,)), ((), ())),
                           precision=_P, preferred_element_type=jnp.float32)


def _elu(x):
    return jnp.where(x > 0, x, jnp.exp(x) - 1.0)


def _tc_mm1(x, w0, w1, w2, w3):
    def body(x_ref, w0_ref, w1_ref, w2_ref, w3_ref, o0, o1, o2, o3):
        xb = x_ref[...]
        o0[...] = _dot(xb, w0_ref[...])
        o1[...] = _dot(xb, w1_ref[...])
        o2[...] = _dot(xb, w2_ref[...])
        o3[...] = _dot(xb, w3_ref[...])

    return pl.pallas_call(
        body,
        grid=(10,),
        in_specs=[pl.BlockSpec((N // 10, D), lambda i: (i, 0))] +
                 [pl.BlockSpec((D, D2), lambda i: (0, 0))] * 4,
        out_specs=[pl.BlockSpec((N // 10, D2), lambda i: (i, 0))] * 4,
        out_shape=[jax.ShapeDtypeStruct((N, D2), jnp.float32)] * 4,
    )(x, w0, w1, w2, w3)


def _tc_mid(accs, cnt_a, cnt_b, w2):
    def body(a0, a1, a2, a3, ca_ref, cb_ref, w_ref, o_ref):
        cnt = ca_ref[:, 0:1] + cb_ref[:, 0:1]
        denom = jnp.maximum(cnt, 1.0)
        acc = _dot(_elu(a0[...] / denom), w_ref[pl.ds(0, D2), :])
        acc += _dot(_elu(a1[...] / denom), w_ref[pl.ds(D2, D2), :])
        acc += _dot(_elu(a2[...] / denom), w_ref[pl.ds(2 * D2, D2), :])
        acc += _dot(_elu(a3[...] / denom), w_ref[pl.ds(3 * D2, D2), :])
        o_ref[...] = acc

    return pl.pallas_call(
        body,
        grid=(10,),
        in_specs=[pl.BlockSpec((NP // 10, D2), lambda i: (i, 0))] * 4 +
                 [pl.BlockSpec((NP // 10, 16), lambda i: (i, 0))] * 2 +
                 [pl.BlockSpec((4 * D2, D2), lambda i: (0, 0))],
        out_specs=[pl.BlockSpec((NP // 10, D2), lambda i: (i, 0))],
        out_shape=[jax.ShapeDtypeStruct((NP, D2), jnp.float32)],
    )(*accs, cnt_a, cnt_b, w2)[0]


def _tc_final(p0, p1, cnt_a, cnt_b):
    def body(p0_ref, p1_ref, ca_ref, cb_ref, o_ref):
        cnt = ca_ref[:, 0:1] + cb_ref[:, 0:1]
        denom = jnp.maximum(cnt, 1.0)
        o_ref[...] = (p0_ref[...] + p1_ref[...]) / denom

    return pl.pallas_call(
        body,
        grid=(10,),
        in_specs=[pl.BlockSpec((NP // 10, D2), lambda i: (i, 0))] * 2 +
                 [pl.BlockSpec((NP // 10, 16), lambda i: (i, 0))] * 2,
        out_specs=[pl.BlockSpec((NP // 10, D2), lambda i: (i, 0))],
        out_shape=[jax.ShapeDtypeStruct((NP, D2), jnp.float32)],
    )(p0, p1, cnt_a, cnt_b)[0]


NBUF = 5
LOOK = 4


def _edge_pipeline(t_hbm, acc_sp, dst_v, src_v, bufs, gsems, ssems, nchunks):
    lag = NBUF - LOOK
    assert nchunks % NBUF == 0
    for j in range(LOOK):
        pltpu.async_copy(t_hbm.at[dst_v.at[j]], bufs[j], gsems[j])

    @pl.loop(0, nchunks // NBUF)
    def _(jj):
        i_base = NBUF * jj
        for b in range(NBUF):
            i = i_base + b
            pltpu.make_async_copy(t_hbm.at[dst_v.at[i]], bufs[b],
                                  gsems[b]).wait()

            @pl.when(i >= lag)
            def _():
                pltpu.make_async_copy(bufs[(b - lag) % NBUF],
                                      acc_sp.at[src_v.at[i - lag]],
                                      ssems[(b - lag) % NBUF]).wait()

            @pl.when(i + LOOK < nchunks)
            def _():
                pltpu.async_copy(t_hbm.at[dst_v.at[i + LOOK]],
                                 bufs[(b + LOOK) % NBUF],
                                 gsems[(b + LOOK) % NBUF])

            pltpu.async_copy(bufs[b], acc_sp.at[src_v.at[i]], ssems[b],
                             add=True)

    for j in range(lag):
        i = nchunks - lag + j
        pltpu.make_async_copy(bufs[i % NBUF], acc_sp.at[src_v.at[i]],
                              ssems[i % NBUF]).wait()


def _zero_vmem(buf, rows, cols):
    @pl.loop(0, rows)
    def _(i):
        @pl.loop(0, cols, step=16)
        def _(j):
            buf[i, pl.ds(j, 16)] = jnp.zeros((16,), jnp.float32)


def _zero_stripe(buf, acc_sp, base):
    @pl.loop(0, RPW // ZROWS)
    def _(k):
        pltpu.sync_copy(buf, acc_sp.at[pl.ds(base + k * ZROWS, ZROWS)])


def _sc_counts(src4d):
    mesh = plsc.VectorSubcoreMesh(core_axis_name="c", subcore_axis_name="s")
    out_type = [jax.ShapeDtypeStruct((NP, 16), jnp.float32)] * 2

    @functools.partial(
        pl.kernel, mesh=mesh, out_type=out_type,
        compiler_params=pltpu.CompilerParams(use_tc_tiling_on_sc=False),
        scratch_types=[
            pltpu.VMEM((CPW_E, CHUNK), jnp.int32),
            pltpu.VMEM((ZROWS, 16), jnp.float32),
            pltpu.VMEM((CHUNK, 16), jnp.float32),
            pltpu.VMEM_SHARED((NP, 16), jnp.float32),
            pltpu.SemaphoreType.DMA,
        ],
    )
    def k(srcc_hbm, cnta_hbm, cntb_hbm, srcc_v, z16_v, ones_v, cnt_sp, csem):
        c = lax.axis_index("c")
        s = lax.axis_index("s")
        base = s * RPW

        pltpu.sync_copy(srcc_hbm.at[c, s], srcc_v)
        _zero_vmem(z16_v, ZROWS, 16)
        _zero_stripe(z16_v, cnt_sp, base)

        @pl.loop(0, CHUNK)
        def _(i):
            ones_v[i, pl.ds(0, 16)] = jnp.ones((16,), jnp.float32)

        plsc.subcore_barrier()

        @pl.loop(0, CPW_E // 8)
        def _(f):
            @pl.loop(0, 8)
            def _(j):
                pltpu.async_copy(ones_v, cnt_sp.at[srcc_v.at[f * 8 + j]],
                                 csem, add=True)

            @pl.loop(0, 8)
            def _(j):
                pltpu.make_async_copy(ones_v, cnt_sp.at[srcc_v.at[f * 8 + j]],
                                      csem).wait()

        plsc.subcore_barrier()

        @pl.when(c == 0)
        def _():
            pltpu.sync_copy(cnt_sp.at[pl.ds(base, RPW)],
                            cnta_hbm.at[pl.ds(base, RPW)])

        @pl.when(c == 1)
        def _():
            pltpu.sync_copy(cnt_sp.at[pl.ds(base, RPW)],
                            cntb_hbm.at[pl.ds(base, RPW)])

    return k(src4d)


def _sc_layer1(t1qs, dst3d, src3d):
    mesh = plsc.VectorSubcoreMesh(core_axis_name="c", subcore_axis_name="s")
    out_type = [jax.ShapeDtypeStruct((NP, D2), jnp.float32)] * 4

    @functools.partial(
        pl.kernel, mesh=mesh, out_type=out_type,
        compiler_params=pltpu.CompilerParams(use_tc_tiling_on_sc=False),
        scratch_types=[
            pltpu.VMEM((CPW_F, CHUNK), jnp.int32),
            pltpu.VMEM((CPW_F, CHUNK), jnp.int32),
        ] + [pltpu.VMEM((CHUNK, D2), jnp.float32)] * NBUF +
        [
            pltpu.VMEM((ZROWS, D2), jnp.float32),
            pltpu.VMEM_SHARED((NP, D2), jnp.float32),
        ] + [pltpu.SemaphoreType.DMA] * (2 * NBUF),
    )
    def k(t1q0_hbm, t1q1_hbm, t1q2_hbm, t1q3_hbm, dst_hbm, src_hbm,
          acc0_hbm, acc1_hbm, acc2_hbm, acc3_hbm,
          dst_v, src_v, *scr):
        bufs = list(scr[:NBUF])
        zf_v = scr[NBUF]
        acc_sp = scr[NBUF + 1]
        gsems = list(scr[NBUF + 2:2 * NBUF + 2])
        ssems = list(scr[2 * NBUF + 2:3 * NBUF + 2])
        c = lax.axis_index("c")
        s = lax.axis_index("s")
        base = s * RPW

        _zero_vmem(zf_v, ZROWS, D2)
        _zero_stripe(zf_v, acc_sp, base)

        plsc.subcore_barrier()

        pltpu.sync_copy(dst_hbm.at[s], dst_v)
        pltpu.sync_copy(src_hbm.at[s], src_v)

        def head_pass(t_hbm, out_hbm):
            _edge_pipeline(t_hbm, acc_sp, dst_v, src_v, bufs, gsems,
                           ssems, CPW_F)
            plsc.subcore_barrier()
            pltpu.sync_copy(acc_sp.at[pl.ds(base, RPW)],
                            out_hbm.at[pl.ds(base, RPW)])

        def core_passes(ta_hbm, tb_hbm, outa_hbm, outb_hbm):
            head_pass(ta_hbm, outa_hbm)
            _zero_stripe(zf_v, acc_sp, base)
            plsc.subcore_barrier()
            head_pass(tb_hbm, outb_hbm)

        @pl.when(c == 0)
        def _():
            core_passes(t1q0_hbm, t1q1_hbm, acc0_hbm, acc1_hbm)

        @pl.when(c == 1)
        def _():
            core_passes(t1q2_hbm, t1q3_hbm, acc2_hbm, acc3_hbm)

    return k(*t1qs, dst3d, src3d)


def _sc_layer2(t2, dst4d, src4d):
    mesh = plsc.VectorSubcoreMesh(core_axis_name="c", subcore_axis_name="s")
    out_type = [
        jax.ShapeDtypeStruct((NP, D2), jnp.float32),
        jax.ShapeDtypeStruct((NP, D2), jnp.float32),
    ]

    @functools.partial(
        pl.kernel, mesh=mesh, out_type=out_type,
        compiler_params=pltpu.CompilerParams(use_tc_tiling_on_sc=False),
        scratch_types=[
            pltpu.VMEM((CPW_E, CHUNK), jnp.int32),
            pltpu.VMEM((CPW_E, CHUNK), jnp.int32),
        ] + [pltpu.VMEM((CHUNK, D2), jnp.float32)] * NBUF +
        [
            pltpu.VMEM((ZROWS, D2), jnp.float32),
            pltpu.VMEM_SHARED((NP, D2), jnp.float32),
        ] + [pltpu.SemaphoreType.DMA] * (2 * NBUF),
    )
    def k(t2_hbm, dst_hbm, src_hbm, p0_hbm, p1_hbm,
          dst_v, src_v, *scr):
        bufs = list(scr[:NBUF])
        zf_v = scr[NBUF]
        acc_sp = scr[NBUF + 1]
        gsems = list(scr[NBUF + 2:2 * NBUF + 2])
        ssems = list(scr[2 * NBUF + 2:3 * NBUF + 2])
        c = lax.axis_index("c")
        s = lax.axis_index("s")
        base = s * RPW

        _zero_vmem(zf_v, ZROWS, D2)
        _zero_stripe(zf_v, acc_sp, base)

        plsc.subcore_barrier()

        pltpu.sync_copy(dst_hbm.at[c, s], dst_v)
        pltpu.sync_copy(src_hbm.at[c, s], src_v)
        _edge_pipeline(t2_hbm, acc_sp, dst_v, src_v, bufs, gsems, ssems,
                       CPW_E)

        plsc.subcore_barrier()

        @pl.when(c == 0)
        def _():
            pltpu.sync_copy(acc_sp.at[pl.ds(base, RPW)],
                            p0_hbm.at[pl.ds(base, RPW)])

        @pl.when(c == 1)
        def _():
            pltpu.sync_copy(acc_sp.at[pl.ds(base, RPW)],
                            p1_hbm.at[pl.ds(base, RPW)])

    return k(t2, dst4d, src4d)


def kernel(node_features, edge_index, W1_0, W1_1, W1_2, W1_3, W2_0):
    src = edge_index[0]
    dst = edge_index[1]
    src3d = src.reshape(NS, CPW_F, CHUNK)
    dst3d = dst.reshape(NS, CPW_F, CHUNK)
    src4d = src.reshape(NC, NS, CPW_E, CHUNK)
    dst4d = dst.reshape(NC, NS, CPW_E, CHUNK)

    cnt_a, cnt_b = _sc_counts(src4d)
    t1qs = _tc_mm1(node_features, W1_0, W1_1, W1_2, W1_3)
    a0, a1, a2, a3 = _sc_layer1(t1qs, dst3d, src3d)
    t2 = _tc_mid((a0, a1, a2, a3), cnt_a, cnt_b, W2_0)
    p0, p1 = _sc_layer2(t2, dst4d, src4d)
    return _tc_final(p0, p1, cnt_a, cnt_b)[:N]


# speedup vs baseline: 1.2480x; 1.0683x over previous
import functools

import jax
import jax.numpy as jnp
from jax import lax
from jax.experimental import pallas as pl
from jax.experimental.pallas import tpu as pltpu
from jax.experimental.pallas import tpu_sc as plsc

N = 10000
E = 320000
D = 128
D2 = 64

NC = 2
NS = 16
CHUNK = 125
ECH = E // CHUNK
CPW_F = ECH // NS
CPW_E = ECH // (NC * NS)
NP = 10240
RPW = NP // NS
ZROWS = 64

_P = lax.Precision.DEFAULT


def _dot(a, b):
    return lax.dot_general(a, b, (((1,), (0,)), ((), ())),
                           precision=_P, preferred_element_type=jnp.float32)


def _elu(x):
    return jnp.where(x > 0, x, jnp.exp(x) - 1.0)


def _tc_mm1(x, w0, w1, w2, w3):
    def body(x_ref, w0_ref, w1_ref, w2_ref, w3_ref, o0, o1, o2, o3):
        xb = x_ref[...]
        o0[...] = _dot(xb, w0_ref[...])
        o1[...] = _dot(xb, w1_ref[...])
        o2[...] = _dot(xb, w2_ref[...])
        o3[...] = _dot(xb, w3_ref[...])

    return pl.pallas_call(
        body,
        grid=(10,),
        in_specs=[pl.BlockSpec((N // 10, D), lambda i: (i, 0))] +
                 [pl.BlockSpec((D, D2), lambda i: (0, 0))] * 4,
        out_specs=[pl.BlockSpec((N // 10, D2), lambda i: (i, 0))] * 4,
        out_shape=[jax.ShapeDtypeStruct((N, D2), jnp.float32)] * 4,
    )(x, w0, w1, w2, w3)


def _tc_mid(accs, cnt_a, cnt_b, w2):
    def body(a0, a1, a2, a3, ca_ref, cb_ref, w_ref, o_ref):
        cnt = ca_ref[:, 0:1] + cb_ref[:, 0:1]
        denom = jnp.maximum(cnt, 1.0)
        acc = _dot(_elu(a0[...] / denom), w_ref[pl.ds(0, D2), :])
        acc += _dot(_elu(a1[...] / denom), w_ref[pl.ds(D2, D2), :])
        acc += _dot(_elu(a2[...] / denom), w_ref[pl.ds(2 * D2, D2), :])
        acc += _dot(_elu(a3[...] / denom), w_ref[pl.ds(3 * D2, D2), :])
        o_ref[...] = acc

    return pl.pallas_call(
        body,
        grid=(10,),
        in_specs=[pl.BlockSpec((NP // 10, D2), lambda i: (i, 0))] * 4 +
                 [pl.BlockSpec((NP // 10, 16), lambda i: (i, 0))] * 2 +
                 [pl.BlockSpec((4 * D2, D2), lambda i: (0, 0))],
        out_specs=[pl.BlockSpec((NP // 10, D2), lambda i: (i, 0))],
        out_shape=[jax.ShapeDtypeStruct((NP, D2), jnp.float32)],
    )(*accs, cnt_a, cnt_b, w2)[0]


def _tc_final(p0, p1, cnt_a, cnt_b):
    def body(p0_ref, p1_ref, ca_ref, cb_ref, o_ref):
        cnt = ca_ref[:, 0:1] + cb_ref[:, 0:1]
        denom = jnp.maximum(cnt, 1.0)
        o_ref[...] = (p0_ref[...] + p1_ref[...]) / denom

    return pl.pallas_call(
        body,
        grid=(10,),
        in_specs=[pl.BlockSpec((N // 10, D2), lambda i: (i, 0))] * 2 +
                 [pl.BlockSpec((N // 10, 16), lambda i: (i, 0))] * 2,
        out_specs=[pl.BlockSpec((N // 10, D2), lambda i: (i, 0))],
        out_shape=[jax.ShapeDtypeStruct((N, D2), jnp.float32)],
    )(p0, p1, cnt_a, cnt_b)[0]


NBUF = 5
LOOK = 4


def _edge_pipeline(t_hbm, acc_sp, dst_v, src_v, bufs, gsems, ssems, nchunks):
    lag = NBUF - LOOK
    assert nchunks % NBUF == 0
    for j in range(LOOK):
        pltpu.async_copy(t_hbm.at[dst_v.at[j]], bufs[j], gsems[j])

    @pl.loop(0, nchunks // NBUF)
    def _(jj):
        i_base = NBUF * jj
        for b in range(NBUF):
            i = i_base + b
            pltpu.make_async_copy(t_hbm.at[dst_v.at[i]], bufs[b],
                                  gsems[b]).wait()

            @pl.when(i >= lag)
            def _():
                pltpu.make_async_copy(bufs[(b - lag) % NBUF],
                                      acc_sp.at[src_v.at[i - lag]],
                                      ssems[(b - lag) % NBUF]).wait()

            @pl.when(i + LOOK < nchunks)
            def _():
                pltpu.async_copy(t_hbm.at[dst_v.at[i + LOOK]],
                                 bufs[(b + LOOK) % NBUF],
                                 gsems[(b + LOOK) % NBUF])

            pltpu.async_copy(bufs[b], acc_sp.at[src_v.at[i]], ssems[b],
                             add=True)

    for j in range(lag):
        i = nchunks - lag + j
        pltpu.make_async_copy(bufs[i % NBUF], acc_sp.at[src_v.at[i]],
                              ssems[i % NBUF]).wait()


def _zero_vmem(buf, rows, cols):
    @pl.loop(0, rows)
    def _(i):
        @pl.loop(0, cols, step=16)
        def _(j):
            buf[i, pl.ds(j, 16)] = jnp.zeros((16,), jnp.float32)


def _zero_stripe(buf, acc_sp, base):
    @pl.loop(0, RPW // ZROWS)
    def _(k):
        pltpu.sync_copy(buf, acc_sp.at[pl.ds(base + k * ZROWS, ZROWS)])


def _sc_counts(er3):
    mesh = plsc.VectorSubcoreMesh(core_axis_name="c", subcore_axis_name="s")
    out_type = [jax.ShapeDtypeStruct((NP, 16), jnp.float32)] * 2

    @functools.partial(
        pl.kernel, mesh=mesh, out_type=out_type,
        compiler_params=pltpu.CompilerParams(use_tc_tiling_on_sc=False),
        scratch_types=[
            pltpu.VMEM((CPW_E, CHUNK), jnp.int32),
            pltpu.VMEM((ZROWS, 16), jnp.float32),
            pltpu.VMEM((CHUNK, 16), jnp.float32),
            pltpu.VMEM_SHARED((NP, 16), jnp.float32),
            pltpu.SemaphoreType.DMA,
        ],
    )
    def k(er_hbm, cnta_hbm, cntb_hbm, srcc_v, z16_v, ones_v, cnt_sp, csem):
        c = lax.axis_index("c")
        s = lax.axis_index("s")
        base = s * RPW

        pltpu.sync_copy(
            er_hbm.at[0, pl.ds(c * (ECH // NC) + s * CPW_E, CPW_E)], srcc_v)
        _zero_vmem(z16_v, ZROWS, 16)
        _zero_stripe(z16_v, cnt_sp, base)

        @pl.loop(0, CHUNK)
        def _(i):
            ones_v[i, pl.ds(0, 16)] = jnp.ones((16,), jnp.float32)

        plsc.subcore_barrier()

        @pl.loop(0, CPW_E // 8)
        def _(f):
            @pl.loop(0, 8)
            def _(j):
                pltpu.async_copy(ones_v, cnt_sp.at[srcc_v.at[f * 8 + j]],
                                 csem, add=True)

            @pl.loop(0, 8)
            def _(j):
                pltpu.make_async_copy(ones_v, cnt_sp.at[srcc_v.at[f * 8 + j]],
                                      csem).wait()

        plsc.subcore_barrier()

        @pl.when(c == 0)
        def _():
            pltpu.sync_copy(cnt_sp.at[pl.ds(base, RPW)],
                            cnta_hbm.at[pl.ds(base, RPW)])

        @pl.when(c == 1)
        def _():
            pltpu.sync_copy(cnt_sp.at[pl.ds(base, RPW)],
                            cntb_hbm.at[pl.ds(base, RPW)])

    return k(er3)


def _sc_layer1(t1qs, er3, cnt_dep):
    mesh = plsc.VectorSubcoreMesh(core_axis_name="c", subcore_axis_name="s")
    out_type = [jax.ShapeDtypeStruct((NP, D2), jnp.float32)] * 4

    @functools.partial(
        pl.kernel, mesh=mesh, out_type=out_type,
        compiler_params=pltpu.CompilerParams(use_tc_tiling_on_sc=False),
        scratch_types=[
            pltpu.VMEM((CPW_F, CHUNK), jnp.int32),
            pltpu.VMEM((CPW_F, CHUNK), jnp.int32),
        ] + [pltpu.VMEM((CHUNK, D2), jnp.float32)] * NBUF +
        [
            pltpu.VMEM((ZROWS, D2), jnp.float32),
            pltpu.VMEM_SHARED((NP, D2), jnp.float32),
        ] + [pltpu.SemaphoreType.DMA] * (2 * NBUF),
    )
    def k(t1q0_hbm, t1q1_hbm, t1q2_hbm, t1q3_hbm, er_hbm, cnt_dep_hbm,
          acc0_hbm, acc1_hbm, acc2_hbm, acc3_hbm,
          dst_v, src_v, *scr):
        del cnt_dep_hbm
        bufs = list(scr[:NBUF])
        zf_v = scr[NBUF]
        acc_sp = scr[NBUF + 1]
        gsems = list(scr[NBUF + 2:2 * NBUF + 2])
        ssems = list(scr[2 * NBUF + 2:3 * NBUF + 2])
        c = lax.axis_index("c")
        s = lax.axis_index("s")
        base = s * RPW

        _zero_vmem(zf_v, ZROWS, D2)
        _zero_stripe(zf_v, acc_sp, base)

        plsc.subcore_barrier()

        pltpu.sync_copy(er_hbm.at[1, pl.ds(s * CPW_F, CPW_F)], dst_v)
        pltpu.sync_copy(er_hbm.at[0, pl.ds(s * CPW_F, CPW_F)], src_v)

        def head_pass(t_hbm, out_hbm):
            _edge_pipeline(t_hbm, acc_sp, dst_v, src_v, bufs, gsems,
                           ssems, CPW_F)
            plsc.subcore_barrier()
            pltpu.sync_copy(acc_sp.at[pl.ds(base, RPW)],
                            out_hbm.at[pl.ds(base, RPW)])

        def core_passes(ta_hbm, tb_hbm, outa_hbm, outb_hbm):
            head_pass(ta_hbm, outa_hbm)
            _zero_stripe(zf_v, acc_sp, base)
            plsc.subcore_barrier()
            head_pass(tb_hbm, outb_hbm)

        @pl.when(c == 0)
        def _():
            core_passes(t1q0_hbm, t1q1_hbm, acc0_hbm, acc1_hbm)

        @pl.when(c == 1)
        def _():
            core_passes(t1q2_hbm, t1q3_hbm, acc2_hbm, acc3_hbm)

    return k(*t1qs, er3, cnt_dep)


def _sc_layer2(t2, er3):
    mesh = plsc.VectorSubcoreMesh(core_axis_name="c", subcore_axis_name="s")
    out_type = [
        jax.ShapeDtypeStruct((NP, D2), jnp.float32),
        jax.ShapeDtypeStruct((NP, D2), jnp.float32),
    ]

    @functools.partial(
        pl.kernel, mesh=mesh, out_type=out_type,
        compiler_params=pltpu.CompilerParams(use_tc_tiling_on_sc=False),
        scratch_types=[
            pltpu.VMEM((CPW_E, CHUNK), jnp.int32),
            pltpu.VMEM((CPW_E, CHUNK), jnp.int32),
        ] + [pltpu.VMEM((CHUNK, D2), jnp.float32)] * NBUF +
        [
            pltpu.VMEM((ZROWS, D2), jnp.float32),
            pltpu.VMEM_SHARED((NP, D2), jnp.float32),
        ] + [pltpu.SemaphoreType.DMA] * (2 * NBUF),
    )
    def k(t2_hbm, er_hbm, p0_hbm, p1_hbm,
          dst_v, src_v, *scr):
        bufs = list(scr[:NBUF])
        zf_v = scr[NBUF]
        acc_sp = scr[NBUF + 1]
        gsems = list(scr[NBUF + 2:2 * NBUF + 2])
        ssems = list(scr[2 * NBUF + 2:3 * NBUF + 2])
        c = lax.axis_index("c")
        s = lax.axis_index("s")
        base = s * RPW

        _zero_vmem(zf_v, ZROWS, D2)
        _zero_stripe(zf_v, acc_sp, base)

        plsc.subcore_barrier()

        row0 = c * (ECH // NC) + s * CPW_E
        pltpu.sync_copy(er_hbm.at[1, pl.ds(row0, CPW_E)], dst_v)
        pltpu.sync_copy(er_hbm.at[0, pl.ds(row0, CPW_E)], src_v)
        _edge_pipeline(t2_hbm, acc_sp, dst_v, src_v, bufs, gsems, ssems,
                       CPW_E)

        plsc.subcore_barrier()

        @pl.when(c == 0)
        def _():
            pltpu.sync_copy(acc_sp.at[pl.ds(base, RPW)],
                            p0_hbm.at[pl.ds(base, RPW)])

        @pl.when(c == 1)
        def _():
            pltpu.sync_copy(acc_sp.at[pl.ds(base, RPW)],
                            p1_hbm.at[pl.ds(base, RPW)])

    return k(t2, er3)


def kernel(node_features, edge_index, W1_0, W1_1, W1_2, W1_3, W2_0):
    er3 = edge_index.reshape(2, ECH, CHUNK)

    cnt_a, cnt_b = _sc_counts(er3)
    t1qs = _tc_mm1(node_features, W1_0, W1_1, W1_2, W1_3)
    a0, a1, a2, a3 = _sc_layer1(t1qs, er3, cnt_a)
    t2 = _tc_mid((a0, a1, a2, a3), cnt_a, cnt_b, W2_0)
    p0, p1 = _sc_layer2(t2, er3)
    return _tc_final(p0, p1, cnt_a, cnt_b)
